# baseline (speedup 1.0000x reference)
import functools

import jax
import jax.numpy as jnp
from jax import lax
from jax.experimental import pallas as pl
from jax.experimental.pallas import tpu as pltpu
from jax.experimental.pallas import tpu_sc as plsc

N_NODES = 100000
D = 128
B = 8192
K = 32

_F32 = jnp.float32
_NW = 32


def _v12_body(k0_ref, k1_ref, aw_ref, o_ref):
    awn = aw_ref[0:1, :D]
    awt = aw_ref[0:1, D:]
    dn = (((1,), (1,)), ((), ()))
    v1t = lax.dot_general(awn, k1_ref[...], dn, preferred_element_type=_F32)
    v2t = lax.dot_general(awt, k0_ref[...], dn, preferred_element_type=_F32)
    o_ref[...] = jnp.concatenate(
        [v1t, v2t, jnp.zeros((6, D), _F32)], axis=0)


def _v12_pass(k0, k1, aw):
    return pl.pallas_call(
        _v12_body,
        out_shape=jax.ShapeDtypeStruct((8, D), _F32),
    )(k0, k1, aw)


_PK_RPW = N_NODES // _NW
_PK_CH = 125
_PK_NCH = _PK_RPW // _PK_CH


def _pack_body(feat, fpk, fio_a, fio_b, sem_a, sem_b):
    nc = plsc.get_sparse_core_info().num_cores
    wid = lax.axis_index("s") * nc + lax.axis_index("c")
    rbase = wid * _PK_RPW

    def issue(ch, fio, sem):
        pltpu.async_copy(feat.at[pl.ds(rbase + ch * _PK_CH, _PK_CH)],
                         fio, sem)

    def pack_chunk(ch, fio, sem):
        pltpu.make_async_copy(feat.at[pl.ds(0, _PK_CH)], fio, sem).wait()

        def rbody(r, carry):
            for c in range(4):
                a = fio[r, pl.ds(32 * c, 16)]
                b = fio[r, pl.ds(32 * c + 16, 16)]
                pk = plsc.pack(a, b, format=plsc.PackFormat.INTERLEAVED)
                fio[r, pl.ds(16 * c, 16)] = plsc.bitcast(pk, _F32)
            return carry

        lax.fori_loop(0, _PK_CH, rbody, 0)
        pltpu.sync_copy(fio.at[:, pl.ds(0, 64)],
                        fpk.at[pl.ds(rbase + ch * _PK_CH, _PK_CH)])

    issue(0, fio_a, sem_a)
    issue(1, fio_b, sem_b)

    def pairbody(it, carry):
        ch = it * 2
        pack_chunk(ch, fio_a, sem_a)
        issue(ch + 2, fio_a, sem_a)
        pack_chunk(ch + 1, fio_b, sem_b)

        @pl.when(it < _PK_NCH // 2 - 1)
        def _():
            issue(ch + 3, fio_b, sem_b)

        return carry

    lax.fori_loop(0, _PK_NCH // 2, pairbody, 0)
    pack_chunk(_PK_NCH - 1, fio_a, sem_a)


def _pack_pass(features):
    mesh = plsc.VectorSubcoreMesh(core_axis_name="c", subcore_axis_name="s")
    fn = functools.partial(
        pl.kernel,
        mesh=mesh,
        compiler_params=pltpu.CompilerParams(
            needs_layout_passes=False, use_tc_tiling_on_sc=False),
        out_type=jax.ShapeDtypeStruct((N_NODES, 64), _F32),
        scratch_types=[
            pltpu.VMEM((_PK_CH, D), _F32),
            pltpu.VMEM((_PK_CH, D), _F32),
            pltpu.SemaphoreType.DMA,
            pltpu.SemaphoreType.DMA,
        ],
    )(_pack_body)
    return fn(features)


_BPW = B // _NW
_IPW = _BPW * K
_NB = 4
_RB = _NB * K
_NBLK = _BPW // _NB
_NBUF = 4
_C = D // 16


def _unpack16(ref, r, c):
    return plsc.unpack(plsc.bitcast(ref[r, pl.ds(16 * c, 16)], jnp.bfloat16),
                       format=plsc.PackFormat.INTERLEAVED)


def _sc_body(fpk, nbr_hbm, node_hbm, v_hbm, agg_hbm,
             idx_v, nidx_v, qrow_v, vv,
             rows_a, rows_b, rows_c, rows_d, agg_v,
             sem_a, sem_b, sem_c, sem_d, sem_q):
    nc = plsc.get_sparse_core_info().num_cores
    wid = lax.axis_index("s") * nc + lax.axis_index("c")
    ibase = wid * _IPW
    nbase = wid * _BPW
    bufs = (rows_a, rows_b, rows_c, rows_d)
    sems = (sem_a, sem_b, sem_c, sem_d)

    pltpu.sync_copy(nbr_hbm.at[pl.ds(ibase, _IPW)], idx_v)
    pltpu.sync_copy(node_hbm.at[pl.ds(nbase, _BPW)], nidx_v)

    def issue(blk, rows_v, sem):
        pltpu.async_copy(fpk.at[idx_v.at[pl.ds(blk * _RB, _RB)]],
                         rows_v, sem)

    def wait(rows_v, sem):
        pltpu.make_async_copy(fpk.at[idx_v.at[pl.ds(0, _RB)]],
                              rows_v, sem).wait()

    for i in range(_NBUF):
        issue(i, bufs[i], sems[i])
    pltpu.async_copy(fpk.at[nidx_v], qrow_v, sem_q)
    pltpu.sync_copy(v_hbm, vv)
    pltpu.make_async_copy(fpk.at[nidx_v], qrow_v, sem_q).wait()

    iota16 = lax.iota(jnp.int32, 16)

    def compute_block(blk, rows_v):
        def nbody(j, carry2):
            b = blk * _NB + j
            r0 = j * K
            v1e = [vv[0, pl.ds(32 * c, 16)] for c in range(4)]
            v1o = [vv[0, pl.ds(32 * c + 16, 16)] for c in range(4)]
            v2e = [vv[1, pl.ds(32 * c, 16)] for c in range(4)]
            v2o = [vv[1, pl.ds(32 * c + 16, 16)] for c in range(4)]

            s0 = jnp.zeros((16,), _F32)
            s1 = jnp.zeros((16,), _F32)
            for k in range(K):
                acc = jnp.zeros((16,), _F32)
                for c in range(4):
                    ae, ao = _unpack16(rows_v, r0 + k, c)
                    acc = acc + ae * v1e[c] + ao * v1o[c]
                sk = jnp.sum(acc)
                if k < 16:
                    s0 = jnp.where(iota16 == k, sk, s0)
                else:
                    s1 = jnp.where(iota16 == (k - 16), sk, s1)

            qacc = jnp.zeros((16,), _F32)
            for c in range(4):
                qe, qo = _unpack16(qrow_v, b, c)
                qacc = qacc + qe * v2e[c] + qo * v2o[c]
            qs = jnp.sum(qacc)

            a0 = s0 + qs
            a1 = s1 + qs
            a0 = jnp.where(a0 >= 0.0, a0, a0 * 0.2)
            a1 = jnp.where(a1 >= 0.0, a1, a1 * 0.2)
            m = jnp.max(jnp.maximum(a0, a1))
            e0 = jnp.exp(a0 - m)
            e1 = jnp.exp(a1 - m)
            den = jnp.broadcast_to(jnp.sum(e0) + jnp.sum(e1), (16,))
            w0 = e0 / den
            w1 = e1 / den

            accs = tuple(jnp.zeros((16,), _F32) for _ in range(_C))
            for k in range(K):
                wk = w0[k] if k < 16 else w1[k - 16]
                new = []
                for c in range(4):
                    ae, ao = _unpack16(rows_v, r0 + k, c)
                    new.append(accs[2 * c] + wk * ae)
                    new.append(accs[2 * c + 1] + wk * ao)
                accs = tuple(new)
            for c in range(4):
                agg_v[j, pl.ds(32 * c, 16)] = accs[2 * c]
                agg_v[j, pl.ds(32 * c + 16, 16)] = accs[2 * c + 1]
            return carry2

        lax.fori_loop(0, _NB, nbody, 0)
        pltpu.sync_copy(agg_v, agg_hbm.at[pl.ds(nbase + blk * _NB, _NB)])

    def ringbody(it, carry):
        blk = it * _NBUF
        for i in range(_NBUF):
            wait(bufs[i], sems[i])
            compute_block(blk + i, bufs[i])
            issue(blk + i + _NBUF, bufs[i], sems[i])
        return carry

    lax.fori_loop(0, _NBLK // _NBUF - 1, ringbody, 0)
    blk = _NBLK - _NBUF
    for i in range(_NBUF):
        wait(bufs[i], sems[i])
        compute_block(blk + i, bufs[i])


def _sc_aggregate(fpk, nbr_flat, node_flat, v12t):
    mesh = plsc.VectorSubcoreMesh(core_axis_name="c", subcore_axis_name="s")
    fn = functools.partial(
        pl.kernel,
        mesh=mesh,
        compiler_params=pltpu.CompilerParams(
            needs_layout_passes=False, use_tc_tiling_on_sc=False),
        out_type=jax.ShapeDtypeStruct((B, D), _F32),
        scratch_types=[
            pltpu.VMEM((_IPW,), jnp.int32),
            pltpu.VMEM((_BPW,), jnp.int32),
            pltpu.VMEM((_BPW, 64), _F32),
            pltpu.VMEM((8, D), _F32),
            pltpu.VMEM((_RB, 64), _F32),
            pltpu.VMEM((_RB, 64), _F32),
            pltpu.VMEM((_RB, 64), _F32),
            pltpu.VMEM((_RB, 64), _F32),
            pltpu.VMEM((_NB, D), _F32),
            pltpu.SemaphoreType.DMA,
            pltpu.SemaphoreType.DMA,
            pltpu.SemaphoreType.DMA,
            pltpu.SemaphoreType.DMA,
            pltpu.SemaphoreType.DMA,
        ],
    )(_sc_body)
    return fn(fpk, nbr_flat, node_flat, v12t)


_MM_ROWS = 2048


def _mm_body(a_ref, k1_ref, nw_ref, o_ref):
    w = jnp.dot(k1_ref[...], nw_ref[...], preferred_element_type=_F32)
    o_ref[...] = jnp.dot(a_ref[...], w, preferred_element_type=_F32)


def _mm_pass(agg, k1, nw):
    return pl.pallas_call(
        _mm_body,
        grid=(B // _MM_ROWS,),
        in_specs=[
            pl.BlockSpec((_MM_ROWS, D), lambda i: (i, 0)),
            pl.BlockSpec((D, D), lambda i: (0, 0)),
            pl.BlockSpec((D, D), lambda i: (0, 0)),
        ],
        out_specs=pl.BlockSpec((_MM_ROWS, D), lambda i: (i, 0)),
        out_shape=jax.ShapeDtypeStruct((B, D), _F32),
    )(agg, k1, nw)


def kernel(features, node, neighbours, attention_weights, kernel, kernel1,
           neigh_weights):
    k0 = kernel.reshape(D, D)
    k1 = kernel1.reshape(D, D)
    v12t = _v12_pass(k0, k1, attention_weights)
    fpk = _pack_pass(features)
    nbr_flat = neighbours.reshape(-1).astype(jnp.int32)
    node_flat = node.reshape(-1).astype(jnp.int32)
    agg = _sc_aggregate(fpk, nbr_flat, node_flat, v12t)
    return _mm_pass(agg, k1, neigh_weights)


# speedup vs baseline: 1.3434x; 1.3434x over previous
import functools

import jax
import jax.numpy as jnp
from jax import lax
from jax.experimental import pallas as pl
from jax.experimental.pallas import tpu as pltpu
from jax.experimental.pallas import tpu_sc as plsc

N_NODES = 100000
D = 128
B = 8192
K = 32

_F32 = jnp.float32


_PQ_ROWS = 4096
_PQ_PAD = _PQ_ROWS * ((N_NODES + _PQ_ROWS - 1) // _PQ_ROWS)


def _pq_body(f_ref, k0_ref, k1_ref, aw_ref, p_ref, q_ref, v12_ref):
    @pl.when(pl.program_id(0) == 0)
    def _():
        awn = aw_ref[0, :D].reshape(D, 1)
        awt = aw_ref[0, D:].reshape(D, 1)
        v1 = jnp.dot(k1_ref[...], awn, preferred_element_type=_F32)
        v2 = jnp.dot(k0_ref[...], awt, preferred_element_type=_F32)
        v12_ref[...] = jnp.concatenate(
            [v1, v2, jnp.zeros((D, 6), _F32)], axis=1)

    pq = jnp.dot(f_ref[...].astype(jnp.bfloat16),
                 v12_ref[...].astype(jnp.bfloat16),
                 preferred_element_type=_F32)
    t = jnp.transpose(pq.reshape(_PQ_ROWS // D, D, 8), (0, 2, 1))
    p_ref[...] = t[:, 0, :]
    q_ref[...] = t[:, 1, :]


def _pq_pass(features, k0, k1, aw):
    return pl.pallas_call(
        _pq_body,
        grid=(pl.cdiv(N_NODES, _PQ_ROWS),),
        in_specs=[
            pl.BlockSpec((_PQ_ROWS, D), lambda i: (i, 0)),
            pl.BlockSpec((D, D), lambda i: (0, 0)),
            pl.BlockSpec((D, D), lambda i: (0, 0)),
            pl.BlockSpec((1, 2 * D), lambda i: (0, 0)),
        ],
        out_specs=[
            pl.BlockSpec((_PQ_ROWS // D, D), lambda i: (i, 0)),
            pl.BlockSpec((_PQ_ROWS // D, D), lambda i: (i, 0)),
        ],
        out_shape=[
            jax.ShapeDtypeStruct((_PQ_PAD // D, D), _F32),
            jax.ShapeDtypeStruct((_PQ_PAD // D, D), _F32),
        ],
        scratch_shapes=[pltpu.VMEM((D, 8), _F32)],
    )(features, k0, k1, aw)


_NW = 32


_BPW = B // _NW
_IPW = _BPW * K
_NB = 4
_RB = _NB * K
_NBLK = _BPW // _NB
_NBUF = 4
_C = D // 16


def _sc_body(feat, p_hbm, q_hbm, nbr_hbm, node_hbm, agg_hbm,
             idx_v, s_v, nidx_v, qv_v,
             rows_a, rows_b, rows_c, rows_d, agg_v,
             sem_a, sem_b, sem_c, sem_d, sem_p, sem_q):
    nc = plsc.get_sparse_core_info().num_cores
    wid = lax.axis_index("s") * nc + lax.axis_index("c")
    ibase = wid * _IPW
    nbase = wid * _BPW
    bufs = (rows_a, rows_b, rows_c, rows_d)
    sems = (sem_a, sem_b, sem_c, sem_d)

    pltpu.sync_copy(nbr_hbm.at[pl.ds(ibase, _IPW)], idx_v)
    pltpu.sync_copy(node_hbm.at[pl.ds(nbase, _BPW)], nidx_v)

    def issue(blk, rows_v, sem):
        pltpu.async_copy(feat.at[idx_v.at[pl.ds(blk * _RB, _RB)]],
                         rows_v, sem)

    def wait(rows_v, sem):
        pltpu.make_async_copy(feat.at[idx_v.at[pl.ds(0, _RB)]],
                              rows_v, sem).wait()

    for i in range(_NBUF):
        issue(i, bufs[i], sems[i])
    chunk = _IPW // 4
    for ch in range(4):
        pltpu.async_copy(p_hbm.at[idx_v.at[pl.ds(ch * chunk, chunk)]],
                         s_v.at[pl.ds(ch * chunk, chunk)], sem_p)
    pltpu.async_copy(q_hbm.at[nidx_v], qv_v, sem_q).wait()

    def wbody(g, carry):
        qv = qv_v[pl.ds(g * 16, 16)]
        for j in range(16):
            base = (g * 16 + j) * K
            qb = qv[j]
            a0 = s_v[pl.ds(base, 16)] + qb
            a1 = s_v[pl.ds(base + 16, 16)] + qb
            a0 = jnp.where(a0 >= 0.0, a0, a0 * 0.2)
            a1 = jnp.where(a1 >= 0.0, a1, a1 * 0.2)
            m = jnp.max(jnp.maximum(a0, a1))
            e0 = jnp.exp(a0 - m)
            e1 = jnp.exp(a1 - m)
            den = jnp.broadcast_to(jnp.sum(e0 + e1), (16,))
            s_v[pl.ds(base, 16)] = e0 / den
            s_v[pl.ds(base + 16, 16)] = e1 / den
        return carry

    groups_per_chunk = _BPW // 16 // 4
    for ch in range(4):
        pltpu.make_async_copy(
            p_hbm.at[idx_v.at[pl.ds(0, chunk)]],
            s_v.at[pl.ds(ch * chunk, chunk)], sem_p).wait()
        lax.fori_loop(ch * groups_per_chunk, (ch + 1) * groups_per_chunk,
                      wbody, 0)

    def compute_block(blk, rows_v):
        def nbody(j, carry2):
            b0 = (blk * _NB + j) * K
            w0 = s_v[pl.ds(b0, 16)]
            w1 = s_v[pl.ds(b0 + 16, 16)]
            accs = tuple(jnp.zeros((16,), _F32) for _ in range(_C))
            for k in range(K):
                wk = w0[k] if k < 16 else w1[k - 16]
                r = j * K + k
                accs = tuple(
                    accs[c] + wk * rows_v[r, pl.ds(c * 16, 16)]
                    for c in range(_C))
            for c in range(_C):
                agg_v[j, pl.ds(c * 16, 16)] = accs[c]
            return carry2

        lax.fori_loop(0, _NB, nbody, 0)
        pltpu.sync_copy(agg_v, agg_hbm.at[pl.ds(nbase + blk * _NB, _NB)])

    def ringbody(it, carry):
        blk = it * _NBUF
        for i in range(_NBUF):
            wait(bufs[i], sems[i])
            compute_block(blk + i, bufs[i])
            issue(blk + i + _NBUF, bufs[i], sems[i])
        return carry

    lax.fori_loop(0, _NBLK // _NBUF - 1, ringbody, 0)
    blk = _NBLK - _NBUF
    for i in range(_NBUF):
        wait(bufs[i], sems[i])
        compute_block(blk + i, bufs[i])


def _sc_aggregate(features, p, q, nbr_flat, node_flat):
    mesh = plsc.VectorSubcoreMesh(core_axis_name="c", subcore_axis_name="s")
    fn = functools.partial(
        pl.kernel,
        mesh=mesh,
        compiler_params=pltpu.CompilerParams(needs_layout_passes=False),
        out_type=jax.ShapeDtypeStruct((B, D), _F32),
        scratch_types=[
            pltpu.VMEM((_IPW,), jnp.int32),
            pltpu.VMEM((_IPW,), _F32),
            pltpu.VMEM((_BPW,), jnp.int32),
            pltpu.VMEM((_BPW,), _F32),
            pltpu.VMEM((_RB, D), _F32),
            pltpu.VMEM((_RB, D), _F32),
            pltpu.VMEM((_RB, D), _F32),
            pltpu.VMEM((_RB, D), _F32),
            pltpu.VMEM((_NB, D), _F32),
            pltpu.SemaphoreType.DMA,
            pltpu.SemaphoreType.DMA,
            pltpu.SemaphoreType.DMA,
            pltpu.SemaphoreType.DMA,
            pltpu.SemaphoreType.DMA,
            pltpu.SemaphoreType.DMA,
        ],
    )(_sc_body)
    return fn(features, p, q, nbr_flat, node_flat)


_MM_ROWS = 2048


def _mm_body(a_ref, k1_ref, nw_ref, o_ref):
    w = jnp.dot(k1_ref[...], nw_ref[...], preferred_element_type=_F32)
    o_ref[...] = jnp.dot(a_ref[...], w, preferred_element_type=_F32)


def _mm_pass(agg, k1, nw):
    return pl.pallas_call(
        _mm_body,
        grid=(B // _MM_ROWS,),
        in_specs=[
            pl.BlockSpec((_MM_ROWS, D), lambda i: (i, 0)),
            pl.BlockSpec((D, D), lambda i: (0, 0)),
            pl.BlockSpec((D, D), lambda i: (0, 0)),
        ],
        out_specs=pl.BlockSpec((_MM_ROWS, D), lambda i: (i, 0)),
        out_shape=jax.ShapeDtypeStruct((B, D), _F32),
    )(agg, k1, nw)


def kernel(features, node, neighbours, attention_weights, kernel, kernel1,
           neigh_weights):
    k0 = kernel.reshape(D, D)
    k1 = kernel1.reshape(D, D)
    p, q = _pq_pass(features, k0, k1, attention_weights)
    p = p.reshape(-1)
    q = q.reshape(-1)
    nbr_flat = neighbours.reshape(-1).astype(jnp.int32)
    node_flat = node.reshape(-1).astype(jnp.int32)
    agg = _sc_aggregate(features, p, q, nbr_flat, node_flat)
    return _mm_pass(agg, k1, neigh_weights)


# speedup vs baseline: 1.4053x; 1.0460x over previous
import functools

import jax
import jax.numpy as jnp
from jax import lax
from jax.experimental import pallas as pl
from jax.experimental.pallas import tpu as pltpu
from jax.experimental.pallas import tpu_sc as plsc

N_NODES = 100000
D = 128
B = 8192
K = 32

_F32 = jnp.float32


_PQ_ROWS = 8192
_PQ_PAD = _PQ_ROWS * ((N_NODES + _PQ_ROWS - 1) // _PQ_ROWS)


def _pq_body(f_ref, k0_ref, k1_ref, aw_ref, p_ref, q_ref, v12_ref):
    @pl.when(pl.program_id(0) == 0)
    def _():
        awn = aw_ref[0, :D].reshape(D, 1)
        awt = aw_ref[0, D:].reshape(D, 1)
        v1 = jnp.dot(k1_ref[...], awn, preferred_element_type=_F32)
        v2 = jnp.dot(k0_ref[...], awt, preferred_element_type=_F32)
        v12_ref[...] = jnp.concatenate(
            [v1, v2, jnp.zeros((D, 6), _F32)], axis=1)

    pq = jnp.dot(f_ref[...].astype(jnp.bfloat16),
                 v12_ref[...].astype(jnp.bfloat16),
                 preferred_element_type=_F32)
    t = jnp.transpose(pq.reshape(_PQ_ROWS // D, D, 8), (0, 2, 1))
    p_ref[...] = t[:, 0, :]
    q_ref[...] = t[:, 1, :]


def _pq_pass(features, k0, k1, aw):
    return pl.pallas_call(
        _pq_body,
        grid=(pl.cdiv(N_NODES, _PQ_ROWS),),
        in_specs=[
            pl.BlockSpec((_PQ_ROWS, D), lambda i: (i, 0)),
            pl.BlockSpec((D, D), lambda i: (0, 0)),
            pl.BlockSpec((D, D), lambda i: (0, 0)),
            pl.BlockSpec((1, 2 * D), lambda i: (0, 0)),
        ],
        out_specs=[
            pl.BlockSpec((_PQ_ROWS // D, D), lambda i: (i, 0)),
            pl.BlockSpec((_PQ_ROWS // D, D), lambda i: (i, 0)),
        ],
        out_shape=[
            jax.ShapeDtypeStruct((_PQ_PAD // D, D), _F32),
            jax.ShapeDtypeStruct((_PQ_PAD // D, D), _F32),
        ],
        scratch_shapes=[pltpu.VMEM((D, 8), _F32)],
    )(features, k0, k1, aw)


_NW = 32


_BPW = B // _NW
_IPW = _BPW * K
_NB = 4
_RB = _NB * K
_NBLK = _BPW // _NB
_NBUF = 4
_C = D // 16


def _sc_body(feat, p_hbm, q_hbm, nbr_hbm, node_hbm, agg_hbm,
             idx_v, s_v, nidx_v, qv_v,
             rows_a, rows_b, rows_c, rows_d, agg_v,
             sem_a, sem_b, sem_c, sem_d, sem_p, sem_q):
    nc = plsc.get_sparse_core_info().num_cores
    wid = lax.axis_index("s") * nc + lax.axis_index("c")
    ibase = wid * _IPW
    nbase = wid * _BPW
    bufs = (rows_a, rows_b, rows_c, rows_d)
    sems = (sem_a, sem_b, sem_c, sem_d)

    pltpu.sync_copy(nbr_hbm.at[pl.ds(ibase, _IPW)], idx_v)
    pltpu.sync_copy(node_hbm.at[pl.ds(nbase, _BPW)], nidx_v)

    def issue(blk, rows_v, sem):
        pltpu.async_copy(feat.at[idx_v.at[pl.ds(blk * _RB, _RB)]],
                         rows_v, sem)

    def wait(rows_v, sem):
        pltpu.make_async_copy(feat.at[idx_v.at[pl.ds(0, _RB)]],
                              rows_v, sem).wait()

    for i in range(_NBUF):
        issue(i, bufs[i], sems[i])
    chunk = _IPW // 4
    for ch in range(4):
        pltpu.async_copy(p_hbm.at[idx_v.at[pl.ds(ch * chunk, chunk)]],
                         s_v.at[pl.ds(ch * chunk, chunk)], sem_p)
    pltpu.async_copy(q_hbm.at[nidx_v], qv_v, sem_q).wait()

    def wbody(g, carry):
        qv = qv_v[pl.ds(g * 16, 16)]
        for j in range(16):
            base = (g * 16 + j) * K
            qb = qv[j]
            a0 = s_v[pl.ds(base, 16)] + qb
            a1 = s_v[pl.ds(base + 16, 16)] + qb
            a0 = jnp.where(a0 >= 0.0, a0, a0 * 0.2)
            a1 = jnp.where(a1 >= 0.0, a1, a1 * 0.2)
            m = jnp.max(jnp.maximum(a0, a1))
            e0 = jnp.exp(a0 - m)
            e1 = jnp.exp(a1 - m)
            den = jnp.broadcast_to(jnp.sum(e0 + e1), (16,))
            s_v[pl.ds(base, 16)] = e0 / den
            s_v[pl.ds(base + 16, 16)] = e1 / den
        return carry

    groups_per_chunk = _BPW // 16 // 4
    for ch in range(4):
        pltpu.make_async_copy(
            p_hbm.at[idx_v.at[pl.ds(0, chunk)]],
            s_v.at[pl.ds(ch * chunk, chunk)], sem_p).wait()
        lax.fori_loop(ch * groups_per_chunk, (ch + 1) * groups_per_chunk,
                      wbody, 0)

    def compute_block(blk, rows_v):
        def nbody(j, carry2):
            b0 = (blk * _NB + j) * K
            w0 = s_v[pl.ds(b0, 16)]
            w1 = s_v[pl.ds(b0 + 16, 16)]
            accs = tuple(jnp.zeros((16,), _F32) for _ in range(_C))
            for k in range(K):
                wk = w0[k] if k < 16 else w1[k - 16]
                r = j * K + k
                accs = tuple(
                    accs[c] + wk * rows_v[r, pl.ds(c * 16, 16)]
                    for c in range(_C))
            for c in range(_C):
                agg_v[j, pl.ds(c * 16, 16)] = accs[c]
            return carry2

        lax.fori_loop(0, _NB, nbody, 0)
        pltpu.sync_copy(agg_v, agg_hbm.at[pl.ds(nbase + blk * _NB, _NB)])

    def ringbody(it, carry):
        blk = it * _NBUF
        for i in range(_NBUF):
            wait(bufs[i], sems[i])
            compute_block(blk + i, bufs[i])
            issue(blk + i + _NBUF, bufs[i], sems[i])
        return carry

    lax.fori_loop(0, _NBLK // _NBUF - 1, ringbody, 0)
    blk = _NBLK - _NBUF
    for i in range(_NBUF):
        wait(bufs[i], sems[i])
        compute_block(blk + i, bufs[i])


def _sc_aggregate(features, p, q, nbr_flat, node_flat):
    mesh = plsc.VectorSubcoreMesh(core_axis_name="c", subcore_axis_name="s")
    fn = functools.partial(
        pl.kernel,
        mesh=mesh,
        compiler_params=pltpu.CompilerParams(needs_layout_passes=False),
        out_type=jax.ShapeDtypeStruct((B, D), _F32),
        scratch_types=[
            pltpu.VMEM((_IPW,), jnp.int32),
            pltpu.VMEM((_IPW,), _F32),
            pltpu.VMEM((_BPW,), jnp.int32),
            pltpu.VMEM((_BPW,), _F32),
            pltpu.VMEM((_RB, D), _F32),
            pltpu.VMEM((_RB, D), _F32),
            pltpu.VMEM((_RB, D), _F32),
            pltpu.VMEM((_RB, D), _F32),
            pltpu.VMEM((_NB, D), _F32),
            pltpu.SemaphoreType.DMA,
            pltpu.SemaphoreType.DMA,
            pltpu.SemaphoreType.DMA,
            pltpu.SemaphoreType.DMA,
            pltpu.SemaphoreType.DMA,
            pltpu.SemaphoreType.DMA,
        ],
    )(_sc_body)
    return fn(features, p, q, nbr_flat, node_flat)


_MM_ROWS = 2048


def _mm_body(a_ref, k1_ref, nw_ref, o_ref):
    w = jnp.dot(k1_ref[...], nw_ref[...], preferred_element_type=_F32)
    o_ref[...] = jnp.dot(a_ref[...], w, preferred_element_type=_F32)


def _mm_pass(agg, k1, nw):
    return pl.pallas_call(
        _mm_body,
        grid=(B // _MM_ROWS,),
        in_specs=[
            pl.BlockSpec((_MM_ROWS, D), lambda i: (i, 0)),
            pl.BlockSpec((D, D), lambda i: (0, 0)),
            pl.BlockSpec((D, D), lambda i: (0, 0)),
        ],
        out_specs=pl.BlockSpec((_MM_ROWS, D), lambda i: (i, 0)),
        out_shape=jax.ShapeDtypeStruct((B, D), _F32),
    )(agg, k1, nw)


def kernel(features, node, neighbours, attention_weights, kernel, kernel1,
           neigh_weights):
    k0 = kernel.reshape(D, D)
    k1 = kernel1.reshape(D, D)
    p, q = _pq_pass(features, k0, k1, attention_weights)
    p = p.reshape(-1)
    q = q.reshape(-1)
    nbr_flat = neighbours.reshape(-1).astype(jnp.int32)
    node_flat = node.reshape(-1).astype(jnp.int32)
    agg = _sc_aggregate(features, p, q, nbr_flat, node_flat)
    return _mm_pass(agg, k1, neigh_weights)


# speedup vs baseline: 1.4219x; 1.0118x over previous
import functools

import jax
import jax.numpy as jnp
from jax import lax
from jax.experimental import pallas as pl
from jax.experimental.pallas import tpu as pltpu
from jax.experimental.pallas import tpu_sc as plsc

N_NODES = 100000
D = 128
B = 8192
K = 32

_F32 = jnp.float32


_PQ_ROWS = 16384
_PQ_PAD = _PQ_ROWS * ((N_NODES + _PQ_ROWS - 1) // _PQ_ROWS)


def _pq_body(f_ref, k0_ref, k1_ref, aw_ref, p_ref, q_ref, v12_ref):
    @pl.when(pl.program_id(0) == 0)
    def _():
        awn = aw_ref[0, :D].reshape(D, 1)
        awt = aw_ref[0, D:].reshape(D, 1)
        v1 = jnp.dot(k1_ref[...], awn, preferred_element_type=_F32)
        v2 = jnp.dot(k0_ref[...], awt, preferred_element_type=_F32)
        v12_ref[...] = jnp.concatenate(
            [v1, v2, jnp.zeros((D, 6), _F32)], axis=1)

    pq = jnp.dot(f_ref[...].astype(jnp.bfloat16),
                 v12_ref[...].astype(jnp.bfloat16),
                 preferred_element_type=_F32)
    t = jnp.transpose(pq.reshape(_PQ_ROWS // D, D, 8), (0, 2, 1))
    p_ref[...] = t[:, 0, :]
    q_ref[...] = t[:, 1, :]


def _pq_pass(features, k0, k1, aw):
    return pl.pallas_call(
        _pq_body,
        grid=(pl.cdiv(N_NODES, _PQ_ROWS),),
        in_specs=[
            pl.BlockSpec((_PQ_ROWS, D), lambda i: (i, 0)),
            pl.BlockSpec((D, D), lambda i: (0, 0)),
            pl.BlockSpec((D, D), lambda i: (0, 0)),
            pl.BlockSpec((1, 2 * D), lambda i: (0, 0)),
        ],
        out_specs=[
            pl.BlockSpec((_PQ_ROWS // D, D), lambda i: (i, 0)),
            pl.BlockSpec((_PQ_ROWS // D, D), lambda i: (i, 0)),
        ],
        out_shape=[
            jax.ShapeDtypeStruct((_PQ_PAD // D, D), _F32),
            jax.ShapeDtypeStruct((_PQ_PAD // D, D), _F32),
        ],
        scratch_shapes=[pltpu.VMEM((D, 8), _F32)],
    )(features, k0, k1, aw)


_NW = 32


_BPW = B // _NW
_IPW = _BPW * K
_NB = 4
_RB = _NB * K
_NBLK = _BPW // _NB
_NBUF = 4
_C = D // 16


def _sc_body(feat, p_hbm, q_hbm, nbr_hbm, node_hbm, agg_hbm,
             idx_v, s_v, nidx_v, qv_v,
             rows_a, rows_b, rows_c, rows_d, agg_v,
             sem_a, sem_b, sem_c, sem_d, sem_p, sem_q):
    nc = plsc.get_sparse_core_info().num_cores
    wid = lax.axis_index("s") * nc + lax.axis_index("c")
    ibase = wid * _IPW
    nbase = wid * _BPW
    bufs = (rows_a, rows_b, rows_c, rows_d)
    sems = (sem_a, sem_b, sem_c, sem_d)

    pltpu.sync_copy(nbr_hbm.at[pl.ds(ibase, _IPW)], idx_v)
    pltpu.sync_copy(node_hbm.at[pl.ds(nbase, _BPW)], nidx_v)

    def issue(blk, rows_v, sem):
        pltpu.async_copy(feat.at[idx_v.at[pl.ds(blk * _RB, _RB)]],
                         rows_v, sem)

    def wait(rows_v, sem):
        pltpu.make_async_copy(feat.at[idx_v.at[pl.ds(0, _RB)]],
                              rows_v, sem).wait()

    for i in range(_NBUF):
        issue(i, bufs[i], sems[i])
    chunk = _IPW // 4
    for ch in range(4):
        pltpu.async_copy(p_hbm.at[idx_v.at[pl.ds(ch * chunk, chunk)]],
                         s_v.at[pl.ds(ch * chunk, chunk)], sem_p)
    pltpu.async_copy(q_hbm.at[nidx_v], qv_v, sem_q).wait()

    def wbody(g, carry):
        qv = qv_v[pl.ds(g * 16, 16)]
        for j in range(16):
            base = (g * 16 + j) * K
            qb = qv[j]
            a0 = s_v[pl.ds(base, 16)] + qb
            a1 = s_v[pl.ds(base + 16, 16)] + qb
            a0 = jnp.where(a0 >= 0.0, a0, a0 * 0.2)
            a1 = jnp.where(a1 >= 0.0, a1, a1 * 0.2)
            m = jnp.max(jnp.maximum(a0, a1))
            e0 = jnp.exp(a0 - m)
            e1 = jnp.exp(a1 - m)
            den = jnp.broadcast_to(jnp.sum(e0 + e1), (16,))
            s_v[pl.ds(base, 16)] = e0 / den
            s_v[pl.ds(base + 16, 16)] = e1 / den
        return carry

    groups_per_chunk = _BPW // 16 // 4
    for ch in range(4):
        pltpu.make_async_copy(
            p_hbm.at[idx_v.at[pl.ds(0, chunk)]],
            s_v.at[pl.ds(ch * chunk, chunk)], sem_p).wait()
        lax.fori_loop(ch * groups_per_chunk, (ch + 1) * groups_per_chunk,
                      wbody, 0)

    def compute_block(blk, rows_v):
        def nbody(j, carry2):
            b0 = (blk * _NB + j) * K
            w0 = s_v[pl.ds(b0, 16)]
            w1 = s_v[pl.ds(b0 + 16, 16)]
            accs = tuple(jnp.zeros((16,), _F32) for _ in range(_C))
            for k in range(K):
                wk = w0[k] if k < 16 else w1[k - 16]
                r = j * K + k
                accs = tuple(
                    accs[c] + wk * rows_v[r, pl.ds(c * 16, 16)]
                    for c in range(_C))
            for c in range(_C):
                agg_v[j, pl.ds(c * 16, 16)] = accs[c]
            return carry2

        lax.fori_loop(0, _NB, nbody, 0)
        pltpu.sync_copy(agg_v, agg_hbm.at[pl.ds(nbase + blk * _NB, _NB)])

    def ringbody(it, carry):
        blk = it * _NBUF
        for i in range(_NBUF):
            wait(bufs[i], sems[i])
            compute_block(blk + i, bufs[i])
            issue(blk + i + _NBUF, bufs[i], sems[i])
        return carry

    lax.fori_loop(0, _NBLK // _NBUF - 1, ringbody, 0)
    blk = _NBLK - _NBUF
    for i in range(_NBUF):
        wait(bufs[i], sems[i])
        compute_block(blk + i, bufs[i])


def _sc_aggregate(features, p, q, nbr_flat, node_flat):
    mesh = plsc.VectorSubcoreMesh(core_axis_name="c", subcore_axis_name="s")
    fn = functools.partial(
        pl.kernel,
        mesh=mesh,
        compiler_params=pltpu.CompilerParams(needs_layout_passes=False),
        out_type=jax.ShapeDtypeStruct((B, D), _F32),
        scratch_types=[
            pltpu.VMEM((_IPW,), jnp.int32),
            pltpu.VMEM((_IPW,), _F32),
            pltpu.VMEM((_BPW,), jnp.int32),
            pltpu.VMEM((_BPW,), _F32),
            pltpu.VMEM((_RB, D), _F32),
            pltpu.VMEM((_RB, D), _F32),
            pltpu.VMEM((_RB, D), _F32),
            pltpu.VMEM((_RB, D), _F32),
            pltpu.VMEM((_NB, D), _F32),
            pltpu.SemaphoreType.DMA,
            pltpu.SemaphoreType.DMA,
            pltpu.SemaphoreType.DMA,
            pltpu.SemaphoreType.DMA,
            pltpu.SemaphoreType.DMA,
            pltpu.SemaphoreType.DMA,
        ],
    )(_sc_body)
    return fn(features, p, q, nbr_flat, node_flat)


_MM_ROWS = 2048


def _mm_body(a_ref, k1_ref, nw_ref, o_ref):
    w = jnp.dot(k1_ref[...], nw_ref[...], preferred_element_type=_F32)
    o_ref[...] = jnp.dot(a_ref[...], w, preferred_element_type=_F32)


def _mm_pass(agg, k1, nw):
    return pl.pallas_call(
        _mm_body,
        grid=(B // _MM_ROWS,),
        in_specs=[
            pl.BlockSpec((_MM_ROWS, D), lambda i: (i, 0)),
            pl.BlockSpec((D, D), lambda i: (0, 0)),
            pl.BlockSpec((D, D), lambda i: (0, 0)),
        ],
        out_specs=pl.BlockSpec((_MM_ROWS, D), lambda i: (i, 0)),
        out_shape=jax.ShapeDtypeStruct((B, D), _F32),
    )(agg, k1, nw)


def kernel(features, node, neighbours, attention_weights, kernel, kernel1,
           neigh_weights):
    k0 = kernel.reshape(D, D)
    k1 = kernel1.reshape(D, D)
    p, q = _pq_pass(features, k0, k1, attention_weights)
    p = p.reshape(-1)
    q = q.reshape(-1)
    nbr_flat = neighbours.reshape(-1).astype(jnp.int32)
    node_flat = node.reshape(-1).astype(jnp.int32)
    agg = _sc_aggregate(features, p, q, nbr_flat, node_flat)
    return _mm_pass(agg, k1, neigh_weights)
